# baseline (speedup 1.0000x reference)
import jax
import jax.numpy as jnp
from jax.experimental import pallas as pl
from jax.experimental.pallas import tpu as pltpu

NUM_ENVS = 16384
NUM_JOINTS = 29
BLOCK_COLS = 1024


def _body(inp_ref, jp_ref, djp_ref, out_ref):
    out_ref[...] = jp_ref[...] - djp_ref[...]

    @pl.when(pl.program_id(0) == 0)
    def _():
        for j in range(NUM_JOINTS):
            out_ref[j : j + 1, 0:1] = inp_ref[j] - djp_ref[j : j + 1, 0:1]


@jax.jit
def _tc_kernel(joint_pos_input, joint_pos, default_joint_pos):
    jp_t = joint_pos.T
    djp_t = default_joint_pos.T
    grid = (NUM_ENVS // BLOCK_COLS,)
    out_t = pl.pallas_call(
        _body,
        grid=grid,
        in_specs=[
            pl.BlockSpec(memory_space=pltpu.SMEM),
            pl.BlockSpec((NUM_JOINTS, BLOCK_COLS), lambda i: (0, i)),
            pl.BlockSpec((NUM_JOINTS, BLOCK_COLS), lambda i: (0, i)),
        ],
        out_specs=pl.BlockSpec((NUM_JOINTS, BLOCK_COLS), lambda i: (0, i)),
        out_shape=jax.ShapeDtypeStruct((NUM_JOINTS, NUM_ENVS), jnp.float32),
    )(joint_pos_input, jp_t, djp_t)
    return out_t.T


def kernel(joint_pos_input, joint_pos, default_joint_pos):
    return _tc_kernel(joint_pos_input, joint_pos, default_joint_pos)


# speedup vs baseline: 2.2761x; 2.2761x over previous
import jax
import jax.numpy as jnp
from jax.experimental import pallas as pl
from jax.experimental.pallas import tpu as pltpu

NUM_ENVS = 16384
NUM_JOINTS = 29
BLOCK_COLS = 4096


def _body(inp_ref, jp_ref, djp_ref, out_ref):
    out_ref[...] = jp_ref[...] - djp_ref[...]

    @pl.when(pl.program_id(0) == 0)
    def _():
        for j in range(NUM_JOINTS):
            out_ref[j : j + 1, 0:1] = inp_ref[j] - djp_ref[j : j + 1, 0:1]


@jax.jit
def _tc_kernel(joint_pos_input, joint_pos, default_joint_pos):
    jp_t = joint_pos.T
    djp_t = default_joint_pos.T
    grid = (NUM_ENVS // BLOCK_COLS,)
    out_t = pl.pallas_call(
        _body,
        grid=grid,
        in_specs=[
            pl.BlockSpec(memory_space=pltpu.SMEM),
            pl.BlockSpec((NUM_JOINTS, BLOCK_COLS), lambda i: (0, i)),
            pl.BlockSpec((NUM_JOINTS, BLOCK_COLS), lambda i: (0, i)),
        ],
        out_specs=pl.BlockSpec((NUM_JOINTS, BLOCK_COLS), lambda i: (0, i)),
        out_shape=jax.ShapeDtypeStruct((NUM_JOINTS, NUM_ENVS), jnp.float32),
    )(joint_pos_input, jp_t, djp_t)
    return out_t.T


def kernel(joint_pos_input, joint_pos, default_joint_pos):
    return _tc_kernel(joint_pos_input, joint_pos, default_joint_pos)


# speedup vs baseline: 3.0505x; 1.3402x over previous
import jax
import jax.numpy as jnp
from jax.experimental import pallas as pl
from jax.experimental.pallas import tpu as pltpu

NUM_ENVS = 16384
NUM_JOINTS = 29
BLOCK_COLS = 8192


def _body(inp_ref, jp_ref, djp_ref, out_ref):
    out_ref[...] = jp_ref[...] - djp_ref[...]

    @pl.when(pl.program_id(0) == 0)
    def _():
        for j in range(NUM_JOINTS):
            out_ref[j : j + 1, 0:1] = inp_ref[j] - djp_ref[j : j + 1, 0:1]


@jax.jit
def _tc_kernel(joint_pos_input, joint_pos, default_joint_pos):
    jp_t = joint_pos.T
    djp_t = default_joint_pos.T
    grid = (NUM_ENVS // BLOCK_COLS,)
    out_t = pl.pallas_call(
        _body,
        grid=grid,
        in_specs=[
            pl.BlockSpec(memory_space=pltpu.SMEM),
            pl.BlockSpec((NUM_JOINTS, BLOCK_COLS), lambda i: (0, i)),
            pl.BlockSpec((NUM_JOINTS, BLOCK_COLS), lambda i: (0, i)),
        ],
        out_specs=pl.BlockSpec((NUM_JOINTS, BLOCK_COLS), lambda i: (0, i)),
        out_shape=jax.ShapeDtypeStruct((NUM_JOINTS, NUM_ENVS), jnp.float32),
    )(joint_pos_input, jp_t, djp_t)
    return out_t.T


def kernel(joint_pos_input, joint_pos, default_joint_pos):
    return _tc_kernel(joint_pos_input, joint_pos, default_joint_pos)
